# Initial kernel scaffold; baseline (speedup 1.0000x reference)
import jax
import jax.numpy as jnp
from jax.experimental import pallas as pl


def kernel(items, nodes, edges, Wi, Wn, We, Wq, Wp):
    raise NotImplementedError("write your pallas kernel here")


# speedup vs baseline: 38.5756x; 38.5756x over previous
import math

import jax
import jax.numpy as jnp
from jax.experimental import pallas as pl
from jax.experimental.pallas import tpu as pltpu

NITEMS = 50
D_ITEM = 16
D_NODE = 16
D_MODEL = 128
FLAG = 15
BSIZE = 64

_NEG = -1e9
_SQRT_D = math.sqrt(float(D_MODEL))


def _dot3(x, w):
    return jax.lax.dot_general(x, w, (((2,), (0,)), ((), ())),
                               preferred_element_type=jnp.float32)


def _dot2(x, w):
    return jax.lax.dot_general(x, w, (((1,), (0,)), ((), ())),
                               preferred_element_type=jnp.float32)


def _bdot(a, v):
    return jax.lax.dot_general(a, v, (((2,), (1,)), ((0,), (0,))),
                               preferred_element_type=jnp.float32)


def _bdot_t(a, b):
    return jax.lax.dot_general(a, b, (((2,), (2,)), ((0,), (0,))),
                               preferred_element_type=jnp.float32)


def _matvec(x, c):
    r = jax.lax.dot_general(x, c[:, :, None], (((2,), (1,)), ((0,), (0,))),
                            preferred_element_type=jnp.float32)
    return r.reshape(BSIZE, NITEMS)


def _lane_sum(e):
    shape = e.shape[:-1]
    pad = jnp.zeros(shape + (128 - e.shape[-1],), jnp.float32)
    y = jnp.concatenate([e, pad], axis=-1)
    k = 128
    while k > 1:
        k //= 2
        y = y[..., :k] + y[..., k:2 * k]
    return y


def _seq_mean(v):
    acc = v[:, 0, :]
    for i in range(1, NITEMS):
        acc = acc + v[:, i, :]
    return acc * jnp.float32(1.0 / NITEMS)


def _rollout_kernel(items_ref, nodes_ref, edges_ref, wi_ref, wn_ref,
                    we_ref, wq_ref, wp_ref,
                    ip_ref, npr_ref, il_ref, nl_ref, sa_ref, pa_ref, rw_ref,
                    vit_ref, s_ref, nodes_cur_ref, avail_ref, selm_ref,
                    fail_ref):
    wi = wi_ref[...]
    wn = wn_ref[...]
    we = we_ref[...]
    wq = wq_ref[...]
    wp = wp_ref[...]
    items = items_ref[...]
    edges = edges_ref[...]

    vitems = jnp.tanh(_dot3(items, wi))
    vit_ref[...] = vitems
    s_ref[...] = _bdot_t(vitems, vitems) / jnp.sqrt(float(D_MODEL))

    nodes_cur_ref[...] = nodes_ref[...]
    avail_ref[...] = jnp.ones((BSIZE, NITEMS, NITEMS), jnp.float32)
    selm_ref[...] = jnp.zeros((BSIZE, NITEMS), jnp.float32)
    fail_ref[...] = jnp.zeros((1, BSIZE), jnp.float32)

    iota_n = jax.lax.broadcasted_iota(jnp.int32, (BSIZE, NITEMS), 1)
    iota_feat = jax.lax.broadcasted_iota(jnp.int32, (BSIZE, D_NODE), 1)

    def step_body(step, carry):
        vit = vit_ref[...]
        nodes_cur = nodes_cur_ref[...]
        avail = avail_ref[...]
        selm = selm_ref[...]

        masked = jnp.where(avail > 0.5, s_ref[...], _NEG)
        m = jnp.max(masked, axis=2, keepdims=True)
        e = jnp.exp(masked - m)
        z = _lane_sum(e)
        attn = e / z
        v2 = vit + _bdot(attn, vit)
        h = jnp.tanh(_dot3(nodes_cur, wn))
        eh = _bdot(edges, h)
        vn = h + _dot3(eh, we)

        ctx = _dot2(_seq_mean(vn), wq)
        isc = _matvec(v2, ctx) / jnp.sqrt(float(D_MODEL))
        isc = jnp.where(selm > 0.5, _NEG, isc)
        im = jnp.max(isc, axis=1, keepdims=True)
        ie = jnp.exp(isc - im)
        iprobs = ie / _lane_sum(ie)
        ip_ref[pl.ds(step, 1)] = iprobs[None]
        ipmax = jnp.max(iprobs, axis=1)
        sel = jnp.min(jnp.where(iprobs == ipmax[:, None], iota_n, NITEMS),
                      axis=1)
        il_ref[pl.ds(step, 1)] = jnp.log(ipmax + 1e-20)[None]
        sa_ref[pl.ds(step, 1)] = sel[None].astype(jnp.int32)
        oh_s = (iota_n == sel[:, None]).astype(jnp.float32)
        selm_ref[...] = jnp.maximum(selm, oh_s)

        keep = (1.0 - oh_s[:, :, None]) * (1.0 - oh_s[:, None, :])
        avail_ref[...] = jnp.maximum(avail * keep,
                                     oh_s[:, :, None] * oh_s[:, None, :])

        vitem = jnp.sum(v2 * oh_s[:, :, None], axis=1)
        q = _dot2(vitem, wp)
        flagcol = nodes_cur[:, :, FLAG]
        nsc = _matvec(vn, q) / jnp.sqrt(float(D_MODEL))
        nsc = jnp.where(flagcol < 0.5, nsc, _NEG)
        nm = jnp.max(nsc, axis=1, keepdims=True)
        ne = jnp.exp(nsc - nm)
        nprobs = ne / _lane_sum(ne)
        npr_ref[pl.ds(step, 1)] = nprobs[None]
        npmax = jnp.max(nprobs, axis=1)
        plc = jnp.min(jnp.where(nprobs == npmax[:, None], iota_n, NITEMS),
                      axis=1)
        nl_ref[pl.ds(step, 1)] = jnp.log(npmax + 1e-20)[None]
        pa_ref[pl.ds(step, 1)] = plc[None].astype(jnp.int32)
        oh_p = (iota_n == plc[:, None]).astype(jnp.float32)

        occ = jnp.sum(oh_p * flagcol, axis=1)
        fail_ref[...] = jnp.maximum(fail_ref[...],
                                    (occ > 0.5).astype(jnp.float32)[None])

        items_sel = jnp.sum(items * oh_s[:, :, None], axis=1)
        newrow = jnp.where(iota_feat == FLAG, 1.0, items_sel)
        nodes_cur_ref[...] = (nodes_cur * (1.0 - oh_p[:, :, None])
                              + oh_p[:, :, None] * newrow[:, None, :])
        return carry

    jax.lax.fori_loop(0, NITEMS, step_body, 0)
    rw_ref[...] = 1.0 - 2.0 * fail_ref[...]


@jax.jit
def kernel(items, nodes, edges, Wi, Wn, We, Wq, Wp):
    f32 = jnp.float32
    out_shapes = [
        jax.ShapeDtypeStruct((NITEMS, BSIZE, NITEMS), f32),
        jax.ShapeDtypeStruct((NITEMS, BSIZE, NITEMS), f32),
        jax.ShapeDtypeStruct((NITEMS, BSIZE), f32),
        jax.ShapeDtypeStruct((NITEMS, BSIZE), f32),
        jax.ShapeDtypeStruct((NITEMS, BSIZE), jnp.int32),
        jax.ShapeDtypeStruct((NITEMS, BSIZE), jnp.int32),
        jax.ShapeDtypeStruct((1, BSIZE), f32),
    ]
    scratch = [
        pltpu.VMEM((BSIZE, NITEMS, D_MODEL), f32),
        pltpu.VMEM((BSIZE, NITEMS, NITEMS), f32),
        pltpu.VMEM((BSIZE, NITEMS, D_NODE), f32),
        pltpu.VMEM((BSIZE, NITEMS, NITEMS), f32),
        pltpu.VMEM((BSIZE, NITEMS), f32),
        pltpu.VMEM((1, BSIZE), f32),
    ]
    ip, npr, il, nl, sa, pa, rw = pl.pallas_call(
        _rollout_kernel,
        out_shape=out_shapes,
        scratch_shapes=scratch,
    )(items, nodes, edges, Wi, Wn, We, Wq, Wp)

    items_probs = jnp.transpose(ip, (1, 0, 2))
    nodes_probs = jnp.transpose(npr, (1, 0, 2))
    items_log_probs = il.T
    nodes_log_probs = nl.T
    all_actions = jnp.stack([sa.T, pa.T], axis=2).reshape(BSIZE, 2 * NITEMS)
    final_rewards = rw.reshape(BSIZE)
    return (items_probs, nodes_probs, items_log_probs, nodes_log_probs,
            all_actions, final_rewards)


# speedup vs baseline: 39.3271x; 1.0195x over previous
import math

import jax
import jax.numpy as jnp
from jax.experimental import pallas as pl
from jax.experimental.pallas import tpu as pltpu

NITEMS = 50
D_ITEM = 16
D_NODE = 16
D_MODEL = 128
FLAG = 15
BSIZE = 64

_NEG = -1e9
_SQRT_D = math.sqrt(float(D_MODEL))


def _dot3(x, w):
    return jax.lax.dot_general(x, w, (((2,), (0,)), ((), ())),
                               preferred_element_type=jnp.float32)


def _dot2(x, w):
    return jax.lax.dot_general(x, w, (((1,), (0,)), ((), ())),
                               preferred_element_type=jnp.float32)


def _bdot(a, v):
    return jax.lax.dot_general(a, v, (((2,), (1,)), ((0,), (0,))),
                               preferred_element_type=jnp.float32)


def _bdot_t(a, b):
    return jax.lax.dot_general(a, b, (((2,), (2,)), ((0,), (0,))),
                               preferred_element_type=jnp.float32)


def _matvec(x, c):
    r = jax.lax.dot_general(x, c[:, :, None], (((2,), (1,)), ((0,), (0,))),
                            preferred_element_type=jnp.float32)
    return r.reshape(BSIZE, NITEMS)


def _lane_sum(e):
    shape = e.shape[:-1]
    pad = jnp.zeros(shape + (64 - e.shape[-1],), jnp.float32)
    y = jnp.concatenate([e, pad], axis=-1)
    k = 64
    while k > 1:
        k //= 2
        y = y[..., :k] + y[..., k:2 * k]
    return y


def _seq_mean(v):
    acc = v[:, 0, :]
    for i in range(1, NITEMS):
        acc = acc + v[:, i, :]
    return acc * jnp.float32(1.0 / NITEMS)


def _rollout_kernel(items_ref, nodes_ref, edges_ref, wi_ref, wn_ref,
                    we_ref, wq_ref, wp_ref,
                    ip_ref, npr_ref, il_ref, nl_ref, sa_ref, pa_ref, rw_ref,
                    vit_ref, s_ref, nodes_cur_ref, selm_ref,
                    fail_ref):
    wi = wi_ref[...]
    wn = wn_ref[...]
    we = we_ref[...]
    wq = wq_ref[...]
    wp = wp_ref[...]
    items = items_ref[...]
    edges = edges_ref[...]

    vitems = jnp.tanh(_dot3(items, wi))
    vit_ref[...] = vitems
    s_ref[...] = _bdot_t(vitems, vitems) / jnp.sqrt(float(D_MODEL))

    nodes_cur_ref[...] = nodes_ref[...]
    selm_ref[...] = jnp.zeros((BSIZE, NITEMS), jnp.float32)
    fail_ref[...] = jnp.zeros((1, BSIZE), jnp.float32)

    iota_n = jax.lax.broadcasted_iota(jnp.int32, (BSIZE, NITEMS), 1)
    iota_feat = jax.lax.broadcasted_iota(jnp.int32, (BSIZE, D_NODE), 1)

    def step_body(step, carry):
        vit = vit_ref[...]
        nodes_cur = nodes_cur_ref[...]
        selm = selm_ref[...]

        ni = 1.0 - selm
        diag = (jax.lax.broadcasted_iota(jnp.int32, (BSIZE, NITEMS, NITEMS), 1)
                == jax.lax.broadcasted_iota(jnp.int32, (BSIZE, NITEMS, NITEMS), 2))
        ok = jnp.logical_or(ni[:, :, None] * ni[:, None, :] > 0.5, diag)
        masked = jnp.where(ok, s_ref[...], _NEG)
        m = jnp.max(masked, axis=2, keepdims=True)
        e = jnp.exp(masked - m)
        z = _lane_sum(e)
        attn = e / z
        v2 = vit + _bdot(attn, vit)
        h = jnp.tanh(_dot3(nodes_cur, wn))
        eh = _bdot(edges, h)
        vn = h + _dot3(eh, we)

        ctx = _dot2(_seq_mean(vn), wq)
        isc = _matvec(v2, ctx) / jnp.sqrt(float(D_MODEL))
        isc = jnp.where(selm > 0.5, _NEG, isc)
        im = jnp.max(isc, axis=1, keepdims=True)
        ie = jnp.exp(isc - im)
        iprobs = ie / _lane_sum(ie)
        ip_ref[pl.ds(step, 1)] = iprobs[None]
        ipmax = jnp.max(iprobs, axis=1)
        sel = jnp.min(jnp.where(iprobs == ipmax[:, None], iota_n, NITEMS),
                      axis=1)
        il_ref[pl.ds(step, 1)] = jnp.log(ipmax + 1e-20)[None]
        sa_ref[pl.ds(step, 1)] = sel[None].astype(jnp.int32)
        oh_s = (iota_n == sel[:, None]).astype(jnp.float32)
        selm_ref[...] = jnp.maximum(selm, oh_s)

        vitem = jnp.sum(v2 * oh_s[:, :, None], axis=1)
        q = _dot2(vitem, wp)
        flagcol = nodes_cur[:, :, FLAG]
        nsc = _matvec(vn, q) / jnp.sqrt(float(D_MODEL))
        nsc = jnp.where(flagcol < 0.5, nsc, _NEG)
        nm = jnp.max(nsc, axis=1, keepdims=True)
        ne = jnp.exp(nsc - nm)
        nprobs = ne / _lane_sum(ne)
        npr_ref[pl.ds(step, 1)] = nprobs[None]
        npmax = jnp.max(nprobs, axis=1)
        plc = jnp.min(jnp.where(nprobs == npmax[:, None], iota_n, NITEMS),
                      axis=1)
        nl_ref[pl.ds(step, 1)] = jnp.log(npmax + 1e-20)[None]
        pa_ref[pl.ds(step, 1)] = plc[None].astype(jnp.int32)
        oh_p = (iota_n == plc[:, None]).astype(jnp.float32)

        occ = jnp.sum(oh_p * flagcol, axis=1)
        fail_ref[...] = jnp.maximum(fail_ref[...],
                                    (occ > 0.5).astype(jnp.float32)[None])

        items_sel = jnp.sum(items * oh_s[:, :, None], axis=1)
        newrow = jnp.where(iota_feat == FLAG, 1.0, items_sel)
        nodes_cur_ref[...] = (nodes_cur * (1.0 - oh_p[:, :, None])
                              + oh_p[:, :, None] * newrow[:, None, :])
        return carry

    jax.lax.fori_loop(0, NITEMS, step_body, 0)
    rw_ref[...] = 1.0 - 2.0 * fail_ref[...]


@jax.jit
def kernel(items, nodes, edges, Wi, Wn, We, Wq, Wp):
    f32 = jnp.float32
    out_shapes = [
        jax.ShapeDtypeStruct((NITEMS, BSIZE, NITEMS), f32),
        jax.ShapeDtypeStruct((NITEMS, BSIZE, NITEMS), f32),
        jax.ShapeDtypeStruct((NITEMS, BSIZE), f32),
        jax.ShapeDtypeStruct((NITEMS, BSIZE), f32),
        jax.ShapeDtypeStruct((NITEMS, BSIZE), jnp.int32),
        jax.ShapeDtypeStruct((NITEMS, BSIZE), jnp.int32),
        jax.ShapeDtypeStruct((1, BSIZE), f32),
    ]
    scratch = [
        pltpu.VMEM((BSIZE, NITEMS, D_MODEL), f32),
        pltpu.VMEM((BSIZE, NITEMS, NITEMS), f32),
        pltpu.VMEM((BSIZE, NITEMS, D_NODE), f32),
        pltpu.VMEM((BSIZE, NITEMS), f32),
        pltpu.VMEM((1, BSIZE), f32),
    ]
    ip, npr, il, nl, sa, pa, rw = pl.pallas_call(
        _rollout_kernel,
        out_shape=out_shapes,
        scratch_shapes=scratch,
    )(items, nodes, edges, Wi, Wn, We, Wq, Wp)

    items_probs = jnp.transpose(ip, (1, 0, 2))
    nodes_probs = jnp.transpose(npr, (1, 0, 2))
    items_log_probs = il.T
    nodes_log_probs = nl.T
    all_actions = jnp.stack([sa.T, pa.T], axis=2).reshape(BSIZE, 2 * NITEMS)
    final_rewards = rw.reshape(BSIZE)
    return (items_probs, nodes_probs, items_log_probs, nodes_log_probs,
            all_actions, final_rewards)
